# baseline (speedup 1.0000x reference)
import functools

import jax
import jax.numpy as jnp
from jax.experimental import pallas as pl
from jax.experimental.pallas import tpu as pltpu
from jax.experimental.pallas import tpu_sc as plsc


def _build_a(cent, k, c, bh):
    crep = jnp.broadcast_to(cent[:, None, :], (k, bh, c))
    crep = crep.reshape(k * bh, c)
    colc = jax.lax.broadcasted_iota(jnp.int32, (c, c * bh), 1)
    rowc = jax.lax.broadcasted_iota(jnp.int32, (c, c * bh), 0)
    selc = (colc // bh == rowc).astype(jnp.float32)
    tmp = jnp.dot(crep, selc,
                  precision=jax.lax.Precision.HIGHEST)
    cola = jax.lax.broadcasted_iota(jnp.int32, (k * bh, c * bh), 1)
    rowa = jax.lax.broadcasted_iota(jnp.int32, (k * bh, c * bh), 0)
    return jnp.where((cola & (bh - 1)) == (rowa & (bh - 1)), tmp, 0.0)


def _make_tc_kernel(w, k, bh, c, h):
    def one_map(f3, a_ref, c2_ref, out_ref, kid):
        for g in range(h // bh):
            f2d = f3[:, g * bh:(g + 1) * bh, :].reshape(c * bh, w)
            cross = jnp.dot(a_ref[...], f2d)
            score = c2_ref[...] - 2.0 * cross
            s3 = score.reshape(k, bh, w)
            smin = jnp.min(s3, axis=0, keepdims=True)
            m = jnp.min(jnp.where(s3 == smin, kid, k), axis=0)
            out_ref[0, g * bh:(g + 1) * bh, :] = m

    def body(f1_ref, f2_ref, cent1_ref, cent2_ref, out1_ref, out2_ref,
             a1_ref, a2_ref, c21_ref, c22_ref):
        i = pl.program_id(0)

        @pl.when(i == 0)
        def _build_constants():
            a1 = _build_a(cent1_ref[...], k, c, bh)
            a1_ref[...] = a1
            c21_ref[...] = jnp.sum(a1 * a1, axis=1, keepdims=True)
            a2 = _build_a(cent2_ref[...], k, c, bh)
            a2_ref[...] = a2
            c22_ref[...] = jnp.sum(a2 * a2, axis=1, keepdims=True)

        kid = jax.lax.broadcasted_iota(jnp.int32, (k, bh, w), 0)
        one_map(f1_ref[0], a1_ref, c21_ref, out1_ref, kid)
        one_map(f2_ref[0], a2_ref, c22_ref, out2_ref, kid)
    return body


def _tc_masks(f1, f2, cent1, cent2):
    b, c, h, w = f1.shape
    k = cent1.shape[0]
    bh = 8
    out_sds = jax.ShapeDtypeStruct((b, h, w), jnp.int32)
    return pl.pallas_call(
        _make_tc_kernel(w, k, bh, c, h),
        grid=(b,),
        in_specs=[
            pl.BlockSpec((1, c, h, w), lambda i: (i, 0, 0, 0)),
            pl.BlockSpec((1, c, h, w), lambda i: (i, 0, 0, 0)),
            pl.BlockSpec((k, c), lambda i: (0, 0)),
            pl.BlockSpec((k, c), lambda i: (0, 0)),
        ],
        out_specs=[
            pl.BlockSpec((1, h, w), lambda i: (i, 0, 0)),
            pl.BlockSpec((1, h, w), lambda i: (i, 0, 0)),
        ],
        out_shape=[out_sds, out_sds],
        scratch_shapes=[
            pltpu.VMEM((k * bh, c * bh), jnp.float32),
            pltpu.VMEM((k * bh, c * bh), jnp.float32),
            pltpu.VMEM((k * bh, 1), jnp.float32),
            pltpu.VMEM((k * bh, 1), jnp.float32),
        ],
    )(f1, f2, cent1, cent2)


def _sc_upsample_two(mask1, mask2, fac):
    r_total, w = mask1.shape
    big_w = w * fac
    info = plsc.get_sparse_core_info()
    nc, ns = info.num_cores, info.num_subcores
    nw = nc * ns
    rpw = r_total // nw
    out_sds = jax.ShapeDtypeStruct((r_total * fac, big_w), jnp.int32)
    mesh = plsc.VectorSubcoreMesh(core_axis_name="c", subcore_axis_name="s")

    @functools.partial(
        pl.kernel, mesh=mesh,
        compiler_params=pltpu.CompilerParams(needs_layout_passes=False),
        out_type=[out_sds, out_sds],
        scratch_types=[
            pltpu.VMEM((w,), jnp.int32),
            pltpu.VMEM((big_w,), jnp.int32),
        ],
    )
    def k(m1_hbm, m2_hbm, o1_hbm, o2_hbm, row_v, exp_v):
        wid = jax.lax.axis_index("s") * nc + jax.lax.axis_index("c")
        base = wid * rpw
        lane = jax.lax.iota(jnp.int32, 16)

        def expand(vi, carry):
            idx = (lane + vi * 16) >> 3
            row_v2 = row_v
            exp_v[pl.ds(vi * 16, 16)] = plsc.load_gather(row_v2, [idx])
            return carry

        for m_hbm, o_hbm in ((m1_hbm, o1_hbm), (m2_hbm, o2_hbm)):
            for rl in range(rpw):
                row = base + rl
                pltpu.sync_copy(m_hbm.at[row], row_v)
                jax.lax.fori_loop(0, big_w // 16, expand, 0)
                for i in range(fac):
                    pltpu.sync_copy(exp_v, o_hbm.at[row * fac + i])

    return k(mask1, mask2)


def kernel(feature_s2t, feature_target, seg_s2t, seg_target, centroid_convert, centroid_target):
    b, c, h, w = feature_s2t.shape
    H, W = seg_s2t.shape[1], seg_s2t.shape[2]
    fac = H // h
    assert H == h * fac and W == w * fac
    m1, m2 = _tc_masks(feature_s2t, feature_target, centroid_target, centroid_convert)
    o1, o2 = _sc_upsample_two(m1.reshape(b * h, w), m2.reshape(b * h, w), fac)
    return (o1.reshape(b, H, W), o2.reshape(b, H, W))


# speedup vs baseline: 1.0410x; 1.0410x over previous
import functools

import jax
import jax.numpy as jnp
from jax.experimental import pallas as pl
from jax.experimental.pallas import tpu as pltpu
from jax.experimental.pallas import tpu_sc as plsc


def _build_a(cent, k, c, bh):
    crep = jnp.broadcast_to(cent[:, None, :], (k, bh, c))
    crep = crep.reshape(k * bh, c)
    colc = jax.lax.broadcasted_iota(jnp.int32, (c, c * bh), 1)
    rowc = jax.lax.broadcasted_iota(jnp.int32, (c, c * bh), 0)
    selc = (colc // bh == rowc).astype(jnp.float32)
    tmp = jnp.dot(crep, selc,
                  precision=jax.lax.Precision.HIGHEST)
    cola = jax.lax.broadcasted_iota(jnp.int32, (k * bh, c * bh), 1)
    rowa = jax.lax.broadcasted_iota(jnp.int32, (k * bh, c * bh), 0)
    return jnp.where((cola & (bh - 1)) == (rowa & (bh - 1)), tmp, 0.0)


def _make_tc_kernel(w, k, bh, c, h):
    def one_map(f3, a_ref, c2_ref, out_ref, kid):
        for g in range(h // bh):
            f2d = f3[:, g * bh:(g + 1) * bh, :].reshape(c * bh, w)
            cross = jnp.dot(a_ref[...], f2d)
            score = c2_ref[...] - 2.0 * cross
            s3 = score.reshape(k, bh, w)
            smin = jnp.min(s3, axis=0, keepdims=True)
            m = jnp.min(jnp.where(s3 == smin, kid, k), axis=0)
            out_ref[0, g * bh:(g + 1) * bh, :] = m

    def body(f1_ref, f2_ref, cent1_ref, cent2_ref, out1_ref, out2_ref,
             a1_ref, a2_ref, c21_ref, c22_ref):
        i = pl.program_id(0)

        @pl.when(i == 0)
        def _build_constants():
            a1 = _build_a(cent1_ref[...], k, c, bh)
            a1_ref[...] = a1
            c21_ref[...] = jnp.sum(a1 * a1, axis=1, keepdims=True)
            a2 = _build_a(cent2_ref[...], k, c, bh)
            a2_ref[...] = a2
            c22_ref[...] = jnp.sum(a2 * a2, axis=1, keepdims=True)

        kid = jax.lax.broadcasted_iota(jnp.int32, (k, bh, w), 0)
        one_map(f1_ref[0], a1_ref, c21_ref, out1_ref, kid)
        one_map(f2_ref[0], a2_ref, c22_ref, out2_ref, kid)
    return body


def _tc_masks(f1, f2, cent1, cent2):
    b, c, h, w = f1.shape
    k = cent1.shape[0]
    bh = 8
    out_sds = jax.ShapeDtypeStruct((b, h, w), jnp.int32)
    return pl.pallas_call(
        _make_tc_kernel(w, k, bh, c, h),
        grid=(b,),
        in_specs=[
            pl.BlockSpec((1, c, h, w), lambda i: (i, 0, 0, 0)),
            pl.BlockSpec((1, c, h, w), lambda i: (i, 0, 0, 0)),
            pl.BlockSpec((k, c), lambda i: (0, 0)),
            pl.BlockSpec((k, c), lambda i: (0, 0)),
        ],
        out_specs=[
            pl.BlockSpec((1, h, w), lambda i: (i, 0, 0)),
            pl.BlockSpec((1, h, w), lambda i: (i, 0, 0)),
        ],
        out_shape=[out_sds, out_sds],
        scratch_shapes=[
            pltpu.VMEM((k * bh, c * bh), jnp.float32),
            pltpu.VMEM((k * bh, c * bh), jnp.float32),
            pltpu.VMEM((k * bh, 1), jnp.float32),
            pltpu.VMEM((k * bh, 1), jnp.float32),
        ],
    )(f1, f2, cent1, cent2)


def _sc_upsample_two(mask1, mask2, fac):
    r_total, w = mask1.shape
    big_w = w * fac
    info = plsc.get_sparse_core_info()
    nc, ns = info.num_cores, info.num_subcores
    nw = nc * ns
    rpw = r_total // nw
    out_sds = jax.ShapeDtypeStruct((r_total * fac, big_w), jnp.int32)
    mesh = plsc.VectorSubcoreMesh(core_axis_name="c", subcore_axis_name="s")

    @functools.partial(
        pl.kernel, mesh=mesh,
        compiler_params=pltpu.CompilerParams(needs_layout_passes=False),
        out_type=[out_sds, out_sds],
        scratch_types=[
            pltpu.VMEM((w,), jnp.int32),
            pltpu.VMEM((fac, big_w), jnp.int32),
        ],
    )
    def k(m1_hbm, m2_hbm, o1_hbm, o2_hbm, row_v, exp_v):
        wid = jax.lax.axis_index("s") * nc + jax.lax.axis_index("c")
        base = wid * rpw
        lane = jax.lax.iota(jnp.int32, 16)

        def expand(vi, carry):
            idx = (lane + vi * 16) >> 3
            vals = plsc.load_gather(row_v, [idx])
            for i in range(fac):
                exp_v[i, pl.ds(vi * 16, 16)] = vals
            return carry

        for m_hbm, o_hbm in ((m1_hbm, o1_hbm), (m2_hbm, o2_hbm)):
            for rl in range(rpw):
                row = base + rl
                pltpu.sync_copy(m_hbm.at[row], row_v)
                jax.lax.fori_loop(0, big_w // 16, expand, 0)
                pltpu.sync_copy(exp_v, o_hbm.at[pl.ds(row * fac, fac)])

    return k(mask1, mask2)


def kernel(feature_s2t, feature_target, seg_s2t, seg_target, centroid_convert, centroid_target):
    b, c, h, w = feature_s2t.shape
    H, W = seg_s2t.shape[1], seg_s2t.shape[2]
    fac = H // h
    assert H == h * fac and W == w * fac
    m1, m2 = _tc_masks(feature_s2t, feature_target, centroid_target, centroid_convert)
    o1, o2 = _sc_upsample_two(m1.reshape(b * h, w), m2.reshape(b * h, w), fac)
    return (o1.reshape(b, H, W), o2.reshape(b, H, W))


# speedup vs baseline: 1.7245x; 1.6566x over previous
import jax
import jax.numpy as jnp
from jax.experimental import pallas as pl
from jax.experimental.pallas import tpu as pltpu


def _build_a(cent, k, c, bh):
    crep = jnp.broadcast_to(cent[:, None, :], (k, bh, c))
    crep = crep.reshape(k * bh, c)
    colc = jax.lax.broadcasted_iota(jnp.int32, (c, c * bh), 1)
    rowc = jax.lax.broadcasted_iota(jnp.int32, (c, c * bh), 0)
    selc = (colc // bh == rowc).astype(jnp.float32)
    tmp = jnp.dot(crep, selc,
                  precision=jax.lax.Precision.HIGHEST)
    cola = jax.lax.broadcasted_iota(jnp.int32, (k * bh, c * bh), 1)
    rowa = jax.lax.broadcasted_iota(jnp.int32, (k * bh, c * bh), 0)
    return jnp.where((cola & (bh - 1)) == (rowa & (bh - 1)), tmp, 0.0)


def _make_kernel(w, k, fac, bh, c, h):
    def one_map(f3, a_ref, c2_ref, sel_ref, out_ref, kid):
        for g in range(h // bh):
            f2d = f3[:, g * bh:(g + 1) * bh, :].reshape(c * bh, w)
            cross = jnp.dot(a_ref[...], f2d)
            score = c2_ref[...] - 2.0 * cross
            s3 = score.reshape(k, bh, w)
            smin = jnp.min(s3, axis=0, keepdims=True)
            m = jnp.min(jnp.where(s3 == smin, kid, k), axis=0)
            mf = m.astype(jnp.float32)
            rep = jnp.dot(mf, sel_ref[...]).astype(jnp.int32)
            rep3 = jnp.broadcast_to(rep[:, None, :], (bh, fac, w * fac))
            out_ref[0, g * bh * fac:(g + 1) * bh * fac, :] = (
                rep3.reshape(bh * fac, w * fac))

    def body(f1_ref, f2_ref, cent1_ref, cent2_ref, out1_ref, out2_ref,
             a1_ref, a2_ref, c21_ref, c22_ref, sel_ref):
        i = pl.program_id(0)

        @pl.when(i == 0)
        def _build_constants():
            a1 = _build_a(cent1_ref[...], k, c, bh)
            a1_ref[...] = a1
            c21_ref[...] = jnp.sum(a1 * a1, axis=1, keepdims=True)
            a2 = _build_a(cent2_ref[...], k, c, bh)
            a2_ref[...] = a2
            c22_ref[...] = jnp.sum(a2 * a2, axis=1, keepdims=True)
            colu = jax.lax.broadcasted_iota(jnp.int32, (w, w * fac), 1)
            rowu = jax.lax.broadcasted_iota(jnp.int32, (w, w * fac), 0)
            sel_ref[...] = (colu // fac == rowu).astype(jnp.float32)

        kid = jax.lax.broadcasted_iota(jnp.int32, (k, bh, w), 0)
        one_map(f1_ref[0], a1_ref, c21_ref, sel_ref, out1_ref, kid)
        one_map(f2_ref[0], a2_ref, c22_ref, sel_ref, out2_ref, kid)
    return body


def kernel(feature_s2t, feature_target, seg_s2t, seg_target, centroid_convert, centroid_target):
    b, c, h, w = feature_s2t.shape
    k = centroid_target.shape[0]
    H, W = seg_s2t.shape[1], seg_s2t.shape[2]
    fac = H // h
    assert H == h * fac and W == w * fac
    assert feature_target.shape == (b, c, h, w)
    assert seg_target.shape[1:] == (H, W)
    bh = 8
    out_sds = jax.ShapeDtypeStruct((b, H, W), jnp.int32)
    mask1, mask2 = pl.pallas_call(
        _make_kernel(w, k, fac, bh, c, h),
        grid=(b,),
        in_specs=[
            pl.BlockSpec((1, c, h, w), lambda i: (i, 0, 0, 0)),
            pl.BlockSpec((1, c, h, w), lambda i: (i, 0, 0, 0)),
            pl.BlockSpec((k, c), lambda i: (0, 0)),
            pl.BlockSpec((k, c), lambda i: (0, 0)),
        ],
        out_specs=[
            pl.BlockSpec((1, H, W), lambda i: (i, 0, 0)),
            pl.BlockSpec((1, H, W), lambda i: (i, 0, 0)),
        ],
        out_shape=[out_sds, out_sds],
        scratch_shapes=[
            pltpu.VMEM((k * bh, c * bh), jnp.float32),
            pltpu.VMEM((k * bh, c * bh), jnp.float32),
            pltpu.VMEM((k * bh, 1), jnp.float32),
            pltpu.VMEM((k * bh, 1), jnp.float32),
            pltpu.VMEM((w, w * fac), jnp.float32),
        ],
    )(feature_s2t, feature_target, centroid_target, centroid_convert)
    return (mask1, mask2)
